# baseline (speedup 1.0000x reference)
import functools

import jax
import jax.numpy as jnp
from jax import lax
from jax.experimental import pallas as pl
from jax.experimental.pallas import tpu as pltpu
from jax.experimental.pallas import tpu_sc as plsc

N = 10000
E = 320000
D = 128
H = 128
EMB = 64
EDGE = 16

NC = 2
NS = 16
CHUNK = 64
EPAD = 327680
ROWS = EPAD // CHUNK
CPT0 = ROWS // NS
CPTC = ROWS // (NC * NS)
NPAD = 10240
RPT = NPAD // NS
RING = 16
NBUF = 5

_f32 = jnp.float32

_mesh = plsc.VectorSubcoreMesh(core_axis_name="c", subcore_axis_name="s")

def _sc_scratch(cpt):
  del cpt
  return [
      pltpu.VMEM_SHARED((NPAD, H), _f32),
      pltpu.VMEM((2, RING, CHUNK), jnp.int32),
      pltpu.VMEM((2, RING, CHUNK), jnp.int32),
      pltpu.VMEM((NBUF, CHUNK, H), _f32),
      pltpu.SemaphoreType.DMA,
      pltpu.SemaphoreType.DMA,
      pltpu.SemaphoreType.DMA,
  ]


def _zero_acc(s, acc, rows, sem):
  zv = jnp.zeros((16,), _f32)
  r0 = rows.at[0]

  def zrow(i, carry):
    for j in range(H // 16):
      r0[i, pl.ds(j * 16, 16)] = zv
    return carry

  lax.fori_loop(0, CHUNK, zrow, 0)
  zs = [pltpu.async_copy(r0, acc.at[pl.ds(s * RPT + k * CHUNK, CHUNK)], sem)
        for k in range(RPT // CHUNK)]
  for d in zs:
    d.wait()


def _edge_sweep(cpt, base0, g_hbm, src_hbm, dst_hbm, acc,
                srcr, dstr, rows, gsem, ssem, isem):
  ngroups = cpt // RING

  pltpu.async_copy(src_hbm.at[pl.ds(base0, RING)], srcr.at[0], isem)
  pltpu.async_copy(dst_hbm.at[pl.ds(base0, RING)], dstr.at[0], isem)

  def group_body(g, carry):
    slot = lax.rem(g, 2)
    sv = srcr.at[slot]
    dv = dstr.at[slot]
    pltpu.make_async_copy(src_hbm.at[pl.ds(base0, RING)], sv, isem).wait()
    pltpu.make_async_copy(dst_hbm.at[pl.ds(base0, RING)], dv, isem).wait()

    @pl.when(g + 1 < ngroups)
    def _():
      nbase = pl.multiple_of(base0 + (g + 1) * RING, RING)
      pltpu.async_copy(src_hbm.at[pl.ds(nbase, RING)], srcr.at[1 - slot],
                       isem)
      pltpu.async_copy(dst_hbm.at[pl.ds(nbase, RING)], dstr.at[1 - slot],
                       isem)

    pend = [None] * NBUF
    gd = [None] * RING
    for j in range(NBUF - 1):
      gd[j] = pltpu.async_copy(g_hbm.at[sv.at[j]], rows.at[j], gsem)
    for j in range(RING):
      b = j % NBUF
      nj = j + NBUF - 1
      if nj < RING:
        nb = nj % NBUF
        if pend[nb] is not None:
          pend[nb].wait()
          pend[nb] = None
        gd[nj] = pltpu.async_copy(g_hbm.at[sv.at[nj]], rows.at[nb], gsem)
      gd[j].wait()
      pend[b] = pltpu.async_copy(rows.at[b], acc.at[dv.at[j]], ssem,
                                 add=True)
    for b in range(NBUF):
      if pend[b] is not None:
        pend[b].wait()
    return carry

  lax.fori_loop(0, ngroups, group_body, 0)


@functools.partial(
    pl.kernel,
    out_type=(
        jax.ShapeDtypeStruct((NPAD, H), _f32),
        jax.ShapeDtypeStruct((NPAD, H), _f32),
    ),
    mesh=_mesh,
    scratch_types=_sc_scratch(CPT0),
)
def _sc_scatter_pair(gu, gi, src_hbm, dst_hbm, su, si,
                     acc, srcr, dstr, rows, gsem, ssem, isem):
  c = lax.axis_index("c")
  s = lax.axis_index("s")

  _zero_acc(s, acc, rows, gsem)
  plsc.subcore_barrier()

  @pl.when(c == 0)
  def _():
    _edge_sweep(CPT0, s * CPT0, gu, src_hbm, dst_hbm, acc,
                srcr, dstr, rows, gsem, ssem, isem)

  @pl.when(c == 1)
  def _():
    _edge_sweep(CPT0, s * CPT0, gi, src_hbm, dst_hbm, acc,
                srcr, dstr, rows, gsem, ssem, isem)

  plsc.subcore_barrier()

  @pl.when(c == 0)
  def _():
    pltpu.sync_copy(acc.at[pl.ds(s * RPT, RPT)], su.at[pl.ds(s * RPT, RPT)])

  @pl.when(c == 1)
  def _():
    pltpu.sync_copy(acc.at[pl.ds(s * RPT, RPT)], si.at[pl.ds(s * RPT, RPT)])


@functools.partial(
    pl.kernel,
    out_type=(
        jax.ShapeDtypeStruct((NPAD, H), _f32),
        jax.ShapeDtypeStruct((NPAD, H), _f32),
    ),
    mesh=_mesh,
    scratch_types=_sc_scratch(CPTC),
)
def _sc_scatter_packed(gc, src_hbm, dst_hbm, p0, p1,
                       acc, srcr, dstr, rows, gsem, ssem, isem):
  c = lax.axis_index("c")
  s = lax.axis_index("s")
  w = c * NS + s

  _zero_acc(s, acc, rows, gsem)
  plsc.subcore_barrier()

  _edge_sweep(CPTC, w * CPTC, gc, src_hbm, dst_hbm, acc,
              srcr, dstr, rows, gsem, ssem, isem)

  plsc.subcore_barrier()

  @pl.when(c == 0)
  def _():
    pltpu.sync_copy(acc.at[pl.ds(s * RPT, RPT)], p0.at[pl.ds(s * RPT, RPT)])

  @pl.when(c == 1)
  def _():
    pltpu.sync_copy(acc.at[pl.ds(s * RPT, RPT)], p1.at[pl.ds(s * RPT, RPT)])


BN = 1000


def _row(width):
  return pl.BlockSpec((BN, width), lambda i: (i, 0))


def _full(a, b):
  return pl.BlockSpec((a, b), lambda i: (0, 0))


def _tc_in_body(xu, xi, wu, wi, ou, oi):
  ou[...] = jnp.dot(xu[...], wu[...], preferred_element_type=_f32)
  oi[...] = jnp.dot(xi[...], wi[...], preferred_element_type=_f32)


def _tc_in(xu, xi, wu, wi):
  return pl.pallas_call(
      _tc_in_body,
      grid=(N // BN,),
      in_specs=[_row(D), _row(D), _full(D, H), _full(D, H)],
      out_specs=[_row(H), _row(H)],
      out_shape=[
          jax.ShapeDtypeStruct((N, H), _f32),
          jax.ShapeDtypeStruct((N, H), _f32),
      ],
  )(xu, xi, wu, wi)


def _tc_mid1_body(su, si, bu, bi, wu, wi, out):
  hu = jnp.maximum(su[...] + bu[...], 0.0)
  hi = jnp.maximum(si[...] + bi[...], 0.0)
  gu = jnp.dot(hu, wu[...], preferred_element_type=_f32)
  gi = jnp.dot(hi, wi[...], preferred_element_type=_f32)
  out[...] = jnp.concatenate([gu, gi], axis=1)


def _tc_mid1(su, si, bu, bi, wu, wi):
  return pl.pallas_call(
      _tc_mid1_body,
      grid=(N // BN,),
      in_specs=[_row(H), _row(H), _full(1, H), _full(1, H),
                _full(H, EMB), _full(H, EMB)],
      out_specs=_row(H),
      out_shape=jax.ShapeDtypeStruct((N, H), _f32),
  )(su, si, bu, bi, wu, wi)


def _tc_midp_body(p0, p1, bc, wbd, out):
  h = jnp.maximum(p0[...] + p1[...] + bc[...], 0.0)
  out[...] = jnp.dot(h, wbd[...], preferred_element_type=_f32)


def _tc_midp(p0, p1, bc, wbd):
  return pl.pallas_call(
      _tc_midp_body,
      grid=(N // BN,),
      in_specs=[_row(H), _row(H), _full(1, H), _full(H, H)],
      out_specs=_row(H),
      out_shape=jax.ShapeDtypeStruct((N, H), _f32),
  )(p0, p1, bc, wbd)


def _tc_head_body(s0u, s0i, p1a, p1b, p2a, p2b, p3a, p3b,
                  bu0, bi0, bc1, bc2, bc3,
                  ef, w0, w1, w2, w3, we, bl, out):
  h0u = jnp.maximum(s0u[...] + bu0[...], 0.0)
  h0i = jnp.maximum(s0i[...] + bi0[...], 0.0)
  acc = jnp.dot(h0u * h0i, w0[...], preferred_element_type=_f32)

  def packed_prod(pa, pb, bc):
    r = jnp.maximum(pa[...] + pb[...] + bc[...], 0.0)
    return r[:, :EMB] * r[:, EMB:]

  acc += jnp.dot(packed_prod(p1a, p1b, bc1), w1[...],
                 preferred_element_type=_f32)
  acc += jnp.dot(packed_prod(p2a, p2b, bc2), w2[...],
                 preferred_element_type=_f32)
  acc += jnp.dot(packed_prod(p3a, p3b, bc3), w3[...],
                 preferred_element_type=_f32)
  acc += jnp.dot(ef[...], we[...], preferred_element_type=_f32)
  out[...] = jax.nn.sigmoid(acc + bl[...])


def _tc_head(s0u, s0i, p1a, p1b, p2a, p2b, p3a, p3b,
             bu0, bi0, bc1, bc2, bc3, ef, w0, w1, w2, w3, we, bl):
  return pl.pallas_call(
      _tc_head_body,
      grid=(N // BN,),
      in_specs=[
          _row(H), _row(H), _row(H), _row(H), _row(H), _row(H),
          _row(H), _row(H),
          _full(1, H), _full(1, H), _full(1, H), _full(1, H), _full(1, H),
          _row(EDGE),
          _full(H, 1), _full(EMB, 1), _full(EMB, 1), _full(EMB, 1),
          _full(EDGE, 1), _full(1, 1),
      ],
      out_specs=pl.BlockSpec((BN, 1), lambda i: (i, 0)),
      out_shape=jax.ShapeDtypeStruct((N, 1), _f32),
  )(s0u, s0i, p1a, p1b, p2a, p2b, p3a, p3b,
    bu0, bi0, bc1, bc2, bc3, ef, w0, w1, w2, w3, we, bl)


def _blockdiag(wu, wi):
  z = jnp.zeros((EMB, EMB), _f32)
  return jnp.concatenate([
      jnp.concatenate([wu, z], axis=1),
      jnp.concatenate([z, wi], axis=1),
  ], axis=0)


def kernel(adjacency, user_feat, item_feat, edge_feature,
           Wu0, bu0, Wu1, bu1, Wu2, bu2, Wu3, bu3,
           Wi0, bi0, Wi1, bi1, Wi2, bi2, Wi3, bi3,
           Wl, bl):
  pad_iota = jnp.arange(EPAD - E, dtype=jnp.int32)
  src2 = jnp.concatenate(
      [adjacency[0].astype(jnp.int32), pad_iota * 37 % N]).reshape(ROWS, CHUNK)
  dst2 = jnp.concatenate(
      [adjacency[1].astype(jnp.int32), N + pad_iota % (NPAD - N)]
  ).reshape(ROWS, CHUNK)

  r = lambda b: b.reshape(1, -1)
  bc1 = jnp.concatenate([bu1, bi1]).reshape(1, H)
  bc2 = jnp.concatenate([bu2, bi2]).reshape(1, H)
  bc3 = jnp.concatenate([bu3, bi3]).reshape(1, H)

  g0u, g0i = _tc_in(user_feat, item_feat, Wu0, Wi0)
  s0u, s0i = _sc_scatter_pair(g0u, g0i, src2, dst2)
  c1 = _tc_mid1(s0u, s0i, r(bu0), r(bi0), Wu1, Wi1)
  p1a, p1b = _sc_scatter_packed(c1, src2, dst2)
  c2 = _tc_midp(p1a, p1b, bc1, _blockdiag(Wu2, Wi2))
  p2a, p2b = _sc_scatter_packed(c2, src2, dst2)
  c3 = _tc_midp(p2a, p2b, bc2, _blockdiag(Wu3, Wi3))
  p3a, p3b = _sc_scatter_packed(c3, src2, dst2)

  out = _tc_head(
      s0u, s0i, p1a, p1b, p2a, p2b, p3a, p3b,
      r(bu0), r(bi0), bc1, bc2, bc3,
      edge_feature,
      Wl[0:H], Wl[H:H + EMB], Wl[H + EMB:H + 2 * EMB],
      Wl[H + 2 * EMB:H + 3 * EMB], Wl[H + 3 * EMB:],
      bl.reshape(1, 1))
  return out


# speedup vs baseline: 1.0365x; 1.0365x over previous
import functools

import jax
import jax.numpy as jnp
from jax import lax
from jax.experimental import pallas as pl
from jax.experimental.pallas import tpu as pltpu
from jax.experimental.pallas import tpu_sc as plsc

N = 10000
E = 320000
D = 128
H = 128
EMB = 64
EDGE = 16

NC = 2
NS = 16
CHUNK = 64
EPAD = 327680
ROWS = EPAD // CHUNK
CPT0 = ROWS // NS
CPTC = ROWS // (NC * NS)
NPAD = 10240
RPT = NPAD // NS
RING = 32
NBUF = 4

_f32 = jnp.float32

_mesh = plsc.VectorSubcoreMesh(core_axis_name="c", subcore_axis_name="s")

def _sc_scratch(cpt):
  del cpt
  return [
      pltpu.VMEM_SHARED((NPAD, H), _f32),
      pltpu.VMEM((2, RING, CHUNK), jnp.int32),
      pltpu.VMEM((2, RING, CHUNK), jnp.int32),
      pltpu.VMEM((NBUF, CHUNK, H), _f32),
      pltpu.SemaphoreType.DMA,
      pltpu.SemaphoreType.DMA,
      pltpu.SemaphoreType.DMA,
  ]


def _zero_acc(s, acc, rows, sem):
  zv = jnp.zeros((16,), _f32)
  r0 = rows.at[0]

  def zrow(i, carry):
    for j in range(H // 16):
      r0[i, pl.ds(j * 16, 16)] = zv
    return carry

  lax.fori_loop(0, CHUNK, zrow, 0)
  zs = [pltpu.async_copy(r0, acc.at[pl.ds(s * RPT + k * CHUNK, CHUNK)], sem)
        for k in range(RPT // CHUNK)]
  for d in zs:
    d.wait()


def _edge_sweep(cpt, base0, g_hbm, src_hbm, dst_hbm, acc,
                srcr, dstr, rows, gsem, ssem, isem):
  ngroups = cpt // RING

  pltpu.async_copy(src_hbm.at[pl.ds(base0, RING)], srcr.at[0], isem)
  pltpu.async_copy(dst_hbm.at[pl.ds(base0, RING)], dstr.at[0], isem)

  def group_body(g, carry):
    slot = lax.rem(g, 2)
    sv = srcr.at[slot]
    dv = dstr.at[slot]
    pltpu.make_async_copy(src_hbm.at[pl.ds(base0, RING)], sv, isem).wait()
    pltpu.make_async_copy(dst_hbm.at[pl.ds(base0, RING)], dv, isem).wait()

    @pl.when(g + 1 < ngroups)
    def _():
      nbase = pl.multiple_of(base0 + (g + 1) * RING, RING)
      pltpu.async_copy(src_hbm.at[pl.ds(nbase, RING)], srcr.at[1 - slot],
                       isem)
      pltpu.async_copy(dst_hbm.at[pl.ds(nbase, RING)], dstr.at[1 - slot],
                       isem)

    pend = [None] * NBUF
    gd = [None] * RING
    for j in range(NBUF - 1):
      gd[j] = pltpu.async_copy(g_hbm.at[sv.at[j]], rows.at[j], gsem)
    for j in range(RING):
      b = j % NBUF
      nj = j + NBUF - 1
      if nj < RING:
        nb = nj % NBUF
        if pend[nb] is not None:
          pend[nb].wait()
          pend[nb] = None
        gd[nj] = pltpu.async_copy(g_hbm.at[sv.at[nj]], rows.at[nb], gsem)
      gd[j].wait()
      pend[b] = pltpu.async_copy(rows.at[b], acc.at[dv.at[j]], ssem,
                                 add=True)
    for b in range(NBUF):
      if pend[b] is not None:
        pend[b].wait()
    return carry

  lax.fori_loop(0, ngroups, group_body, 0)


@functools.partial(
    pl.kernel,
    out_type=(
        jax.ShapeDtypeStruct((NPAD, H), _f32),
        jax.ShapeDtypeStruct((NPAD, H), _f32),
    ),
    mesh=_mesh,
    scratch_types=_sc_scratch(CPT0),
)
def _sc_scatter_pair(gu, gi, src_hbm, dst_hbm, su, si,
                     acc, srcr, dstr, rows, gsem, ssem, isem):
  c = lax.axis_index("c")
  s = lax.axis_index("s")

  _zero_acc(s, acc, rows, gsem)
  plsc.subcore_barrier()

  @pl.when(c == 0)
  def _():
    _edge_sweep(CPT0, s * CPT0, gu, src_hbm, dst_hbm, acc,
                srcr, dstr, rows, gsem, ssem, isem)

  @pl.when(c == 1)
  def _():
    _edge_sweep(CPT0, s * CPT0, gi, src_hbm, dst_hbm, acc,
                srcr, dstr, rows, gsem, ssem, isem)

  plsc.subcore_barrier()

  @pl.when(c == 0)
  def _():
    pltpu.sync_copy(acc.at[pl.ds(s * RPT, RPT)], su.at[pl.ds(s * RPT, RPT)])

  @pl.when(c == 1)
  def _():
    pltpu.sync_copy(acc.at[pl.ds(s * RPT, RPT)], si.at[pl.ds(s * RPT, RPT)])


@functools.partial(
    pl.kernel,
    out_type=(
        jax.ShapeDtypeStruct((NPAD, H), _f32),
        jax.ShapeDtypeStruct((NPAD, H), _f32),
    ),
    mesh=_mesh,
    scratch_types=_sc_scratch(CPTC),
)
def _sc_scatter_packed(gc, src_hbm, dst_hbm, p0, p1,
                       acc, srcr, dstr, rows, gsem, ssem, isem):
  c = lax.axis_index("c")
  s = lax.axis_index("s")
  w = c * NS + s

  _zero_acc(s, acc, rows, gsem)
  plsc.subcore_barrier()

  _edge_sweep(CPTC, w * CPTC, gc, src_hbm, dst_hbm, acc,
              srcr, dstr, rows, gsem, ssem, isem)

  plsc.subcore_barrier()

  @pl.when(c == 0)
  def _():
    pltpu.sync_copy(acc.at[pl.ds(s * RPT, RPT)], p0.at[pl.ds(s * RPT, RPT)])

  @pl.when(c == 1)
  def _():
    pltpu.sync_copy(acc.at[pl.ds(s * RPT, RPT)], p1.at[pl.ds(s * RPT, RPT)])


BN = 1000


def _row(width):
  return pl.BlockSpec((BN, width), lambda i: (i, 0))


def _full(a, b):
  return pl.BlockSpec((a, b), lambda i: (0, 0))


def _tc_in_body(xu, xi, wu, wi, ou, oi):
  ou[...] = jnp.dot(xu[...], wu[...], preferred_element_type=_f32)
  oi[...] = jnp.dot(xi[...], wi[...], preferred_element_type=_f32)


def _tc_in(xu, xi, wu, wi):
  return pl.pallas_call(
      _tc_in_body,
      grid=(N // BN,),
      in_specs=[_row(D), _row(D), _full(D, H), _full(D, H)],
      out_specs=[_row(H), _row(H)],
      out_shape=[
          jax.ShapeDtypeStruct((N, H), _f32),
          jax.ShapeDtypeStruct((N, H), _f32),
      ],
  )(xu, xi, wu, wi)


def _tc_mid1_body(su, si, bu, bi, wu, wi, out):
  hu = jnp.maximum(su[...] + bu[...], 0.0)
  hi = jnp.maximum(si[...] + bi[...], 0.0)
  gu = jnp.dot(hu, wu[...], preferred_element_type=_f32)
  gi = jnp.dot(hi, wi[...], preferred_element_type=_f32)
  out[...] = jnp.concatenate([gu, gi], axis=1)


def _tc_mid1(su, si, bu, bi, wu, wi):
  return pl.pallas_call(
      _tc_mid1_body,
      grid=(N // BN,),
      in_specs=[_row(H), _row(H), _full(1, H), _full(1, H),
                _full(H, EMB), _full(H, EMB)],
      out_specs=_row(H),
      out_shape=jax.ShapeDtypeStruct((N, H), _f32),
  )(su, si, bu, bi, wu, wi)


def _tc_midp_body(p0, p1, bc, wbd, out):
  h = jnp.maximum(p0[...] + p1[...] + bc[...], 0.0)
  out[...] = jnp.dot(h, wbd[...], preferred_element_type=_f32)


def _tc_midp(p0, p1, bc, wbd):
  return pl.pallas_call(
      _tc_midp_body,
      grid=(N // BN,),
      in_specs=[_row(H), _row(H), _full(1, H), _full(H, H)],
      out_specs=_row(H),
      out_shape=jax.ShapeDtypeStruct((N, H), _f32),
  )(p0, p1, bc, wbd)


def _tc_head_body(s0u, s0i, p1a, p1b, p2a, p2b, p3a, p3b,
                  bu0, bi0, bc1, bc2, bc3,
                  ef, w0, w1, w2, w3, we, bl, out):
  h0u = jnp.maximum(s0u[...] + bu0[...], 0.0)
  h0i = jnp.maximum(s0i[...] + bi0[...], 0.0)
  acc = jnp.dot(h0u * h0i, w0[...], preferred_element_type=_f32)

  def packed_prod(pa, pb, bc):
    r = jnp.maximum(pa[...] + pb[...] + bc[...], 0.0)
    return r[:, :EMB] * r[:, EMB:]

  acc += jnp.dot(packed_prod(p1a, p1b, bc1), w1[...],
                 preferred_element_type=_f32)
  acc += jnp.dot(packed_prod(p2a, p2b, bc2), w2[...],
                 preferred_element_type=_f32)
  acc += jnp.dot(packed_prod(p3a, p3b, bc3), w3[...],
                 preferred_element_type=_f32)
  acc += jnp.dot(ef[...], we[...], preferred_element_type=_f32)
  out[...] = jax.nn.sigmoid(acc + bl[...])


def _tc_head(s0u, s0i, p1a, p1b, p2a, p2b, p3a, p3b,
             bu0, bi0, bc1, bc2, bc3, ef, w0, w1, w2, w3, we, bl):
  return pl.pallas_call(
      _tc_head_body,
      grid=(N // BN,),
      in_specs=[
          _row(H), _row(H), _row(H), _row(H), _row(H), _row(H),
          _row(H), _row(H),
          _full(1, H), _full(1, H), _full(1, H), _full(1, H), _full(1, H),
          _row(EDGE),
          _full(H, 1), _full(EMB, 1), _full(EMB, 1), _full(EMB, 1),
          _full(EDGE, 1), _full(1, 1),
      ],
      out_specs=pl.BlockSpec((BN, 1), lambda i: (i, 0)),
      out_shape=jax.ShapeDtypeStruct((N, 1), _f32),
  )(s0u, s0i, p1a, p1b, p2a, p2b, p3a, p3b,
    bu0, bi0, bc1, bc2, bc3, ef, w0, w1, w2, w3, we, bl)


def _blockdiag(wu, wi):
  z = jnp.zeros((EMB, EMB), _f32)
  return jnp.concatenate([
      jnp.concatenate([wu, z], axis=1),
      jnp.concatenate([z, wi], axis=1),
  ], axis=0)


def kernel(adjacency, user_feat, item_feat, edge_feature,
           Wu0, bu0, Wu1, bu1, Wu2, bu2, Wu3, bu3,
           Wi0, bi0, Wi1, bi1, Wi2, bi2, Wi3, bi3,
           Wl, bl):
  pad_iota = jnp.arange(EPAD - E, dtype=jnp.int32)
  src2 = jnp.concatenate(
      [adjacency[0].astype(jnp.int32), pad_iota * 37 % N]).reshape(ROWS, CHUNK)
  dst2 = jnp.concatenate(
      [adjacency[1].astype(jnp.int32), N + pad_iota % (NPAD - N)]
  ).reshape(ROWS, CHUNK)

  r = lambda b: b.reshape(1, -1)
  bc1 = jnp.concatenate([bu1, bi1]).reshape(1, H)
  bc2 = jnp.concatenate([bu2, bi2]).reshape(1, H)
  bc3 = jnp.concatenate([bu3, bi3]).reshape(1, H)

  g0u, g0i = _tc_in(user_feat, item_feat, Wu0, Wi0)
  s0u, s0i = _sc_scatter_pair(g0u, g0i, src2, dst2)
  c1 = _tc_mid1(s0u, s0i, r(bu0), r(bi0), Wu1, Wi1)
  p1a, p1b = _sc_scatter_packed(c1, src2, dst2)
  c2 = _tc_midp(p1a, p1b, bc1, _blockdiag(Wu2, Wi2))
  p2a, p2b = _sc_scatter_packed(c2, src2, dst2)
  c3 = _tc_midp(p2a, p2b, bc2, _blockdiag(Wu3, Wi3))
  p3a, p3b = _sc_scatter_packed(c3, src2, dst2)

  out = _tc_head(
      s0u, s0i, p1a, p1b, p2a, p2b, p3a, p3b,
      r(bu0), r(bi0), bc1, bc2, bc3,
      edge_feature,
      Wl[0:H], Wl[H:H + EMB], Wl[H + EMB:H + 2 * EMB],
      Wl[H + 2 * EMB:H + 3 * EMB], Wl[H + 3 * EMB:],
      bl.reshape(1, 1))
  return out
